# baseline (speedup 1.0000x reference)
import functools

import jax
import jax.numpy as jnp
from jax import lax
from jax.experimental import pallas as pl
from jax.experimental.pallas import tpu as pltpu
from jax.experimental.pallas import tpu_sc as plsc

N, E, FN, FE, H = 10000, 320000, 128, 16, 128
NC, NS = 2, 16
NW = NC * NS
CH = 128
CHUNKS = E // CH
CPW = 80
LASTW = CHUNKS - (NW - 1) * CPW
PADC = 4
LASTW_LD = LASTW + PADC
NPAD = 10240
NPT = NPAD // NS
BE = 8000
BN = 1000
BNP = 1024
GG = 2
ROWS_G = GG * CH


def _sc_mesh():
    return plsc.VectorSubcoreMesh(core_axis_name="c", subcore_axis_name="s")


def _tc_pairflip(ei3):
    def body(x_ref, o_ref):
        x = x_ref[...]
        lane = lax.broadcasted_iota(jnp.int32, x.shape, 2)
        o_ref[...] = jnp.where((lane & 1) == 0,
                               pltpu.roll(x, CH - 1, 2),
                               pltpu.roll(x, 1, 2))

    return pl.pallas_call(
        body,
        out_shape=jax.ShapeDtypeStruct((2, CHUNKS + PADC, CH), jnp.int32),
    )(ei3)


def _sc_gather(table, idxarr, which):
    @functools.partial(
        pl.kernel,
        out_type=jax.ShapeDtypeStruct((E, H), jnp.float32),
        mesh=_sc_mesh(),
        scratch_types=[
            pltpu.VMEM((CPW, CH), jnp.int32),
            pltpu.VMEM((2, ROWS_G, H), jnp.float32),
            pltpu.SemaphoreType.DMA,
            pltpu.SemaphoreType.DMA,
            pltpu.SemaphoreType.DMA,
        ],
    )
    def k(table_hbm, idx_hbm, out_hbm, idxs_v, bufs, sg, ss0, ss1):
        cid = lax.axis_index("c")
        sid = lax.axis_index("s")
        wid = sid * NC + cid
        cb = wid * CPW
        base = cb * CH
        last = wid == NW - 1

        @pl.when(last)
        def _():
            pltpu.sync_copy(idx_hbm.at[which, pl.ds(cb, LASTW_LD)],
                            idxs_v.at[pl.ds(0, LASTW_LD)])

        @pl.when(jnp.logical_not(last))
        def _():
            pltpu.sync_copy(idx_hbm.at[which, pl.ds(cb, CPW)], idxs_v)

        half_groups = jnp.where(last, LASTW // GG // 2, CPW // GG // 2)
        ss = (ss0, ss1)

        def run_group(g, s):
            buf = bufs.at[s]
            for b in range(GG):
                pltpu.async_copy(table_hbm.at[idxs_v.at[g * GG + b]],
                                 buf.at[pl.ds(b * CH, CH)], sg)
            for b in range(GG):
                pltpu.make_async_copy(table_hbm.at[idxs_v.at[0]],
                                      buf.at[pl.ds(b * CH, CH)], sg).wait()
            pltpu.async_copy(buf, out_hbm.at[pl.ds(base + g * ROWS_G, ROWS_G)],
                             ss[s])

        def drain_store(s):
            pltpu.make_async_copy(bufs.at[s],
                                  out_hbm.at[pl.ds(base, ROWS_G)], ss[s]).wait()

        def body(i, carry):
            for s in range(2):
                @pl.when(i >= 1)
                def _():
                    drain_store(s)

                run_group(2 * i + s, s)
            return carry

        lax.fori_loop(0, half_groups, body, 0)
        drain_store(0)
        drain_store(1)

    return k(table, idxarr)


def _sc_segsum(h, idxarr, which, zeros):
    @functools.partial(
        pl.kernel,
        out_type=jax.ShapeDtypeStruct((NC, NPAD, H), jnp.float32),
        mesh=_sc_mesh(),
        scratch_types=[
            pltpu.VMEM((CPW, CH), jnp.int32),
            pltpu.VMEM((2, CH, H), jnp.float32),
            pltpu.VMEM_SHARED((NPAD, H), jnp.float32),
            pltpu.SemaphoreType.DMA,
            pltpu.SemaphoreType.DMA,
            pltpu.SemaphoreType.DMA,
        ],
    )
    def k(h_hbm, idx_hbm, z_hbm, out_hbm, idxs_v, bufs, acc, sl, sa0, sa1):
        cid = lax.axis_index("c")
        sid = lax.axis_index("s")
        wid = sid * NC + cid
        cb = wid * CPW
        base = cb * CH
        last = wid == NW - 1
        pltpu.sync_copy(z_hbm, acc.at[pl.ds(sid * NPT, NPT)])

        @pl.when(last)
        def _():
            pltpu.sync_copy(idx_hbm.at[which, pl.ds(cb, LASTW_LD)],
                            idxs_v.at[pl.ds(0, LASTW_LD)])

        @pl.when(jnp.logical_not(last))
        def _():
            pltpu.sync_copy(idx_hbm.at[which, pl.ds(cb, CPW)], idxs_v)

        half_chunks = jnp.where(last, LASTW // 2, CPW // 2)
        plsc.subcore_barrier()
        sa = (sa0, sa1)

        def drain_add(s):
            pltpu.make_async_copy(bufs.at[s], acc.at[idxs_v.at[0]],
                                  sa[s]).wait()

        def run_chunk(j, s):
            buf = bufs.at[s]
            pltpu.async_copy(h_hbm.at[pl.ds(base + j * CH, CH)], buf, sl)
            pltpu.make_async_copy(h_hbm.at[pl.ds(base, CH)], buf, sl).wait()
            pltpu.async_copy(buf, acc.at[idxs_v.at[j]], sa[s], add=True)

        def body(i, carry):
            for s in range(2):
                @pl.when(i >= 1)
                def _():
                    drain_add(s)

                run_chunk(2 * i + s, s)
            return carry

        lax.fori_loop(0, half_chunks, body, 0)
        drain_add(0)
        drain_add(1)
        plsc.subcore_barrier()
        pltpu.sync_copy(acc.at[pl.ds(sid * NPT, NPT)],
                        out_hbm.at[cid, pl.ds(sid * NPT, NPT)])

    return k(h, idxarr, zeros)


def _tc_mm_node(x, w):
    def body(x_ref, w_ref, o_ref):
        o_ref[...] = jnp.dot(x_ref[...], w_ref[...],
                             preferred_element_type=jnp.float32)

    return pl.pallas_call(
        body,
        grid=(N // BN,),
        in_specs=[pl.BlockSpec((BN, FN), lambda i: (i, 0)),
                  pl.BlockSpec((FN, H), lambda i: (0, 0))],
        out_specs=pl.BlockSpec((BN, H), lambda i: (i, 0)),
        out_shape=jax.ShapeDtypeStruct((N, H), jnp.float32),
    )(x, w)


def _tc_aggw(p, w, b):
    def body(p_ref, w_ref, b_ref, o_ref):
        agg = p_ref[0] + p_ref[1]
        o_ref[...] = jnp.dot(agg, w_ref[...],
                             preferred_element_type=jnp.float32) + b_ref[...]

    return pl.pallas_call(
        body,
        grid=(NPAD // BNP,),
        in_specs=[pl.BlockSpec((NC, BNP, H), lambda i: (0, i, 0)),
                  pl.BlockSpec((H, H), lambda i: (0, 0)),
                  pl.BlockSpec((1, H), lambda i: (0, 0))],
        out_specs=pl.BlockSpec((BNP, H), lambda i: (i, 0)),
        out_shape=jax.ShapeDtypeStruct((NPAD, H), jnp.float32),
    )(p, w, b)


def _tc_init_h(gath, ea, w2, b):
    def body(g_ref, e_ref, w_ref, b_ref, o_ref):
        o_ref[...] = jnp.maximum(
            g_ref[...] + jnp.dot(e_ref[...], w_ref[...],
                                 preferred_element_type=jnp.float32)
            + b_ref[...], 0.0)

    return pl.pallas_call(
        body,
        grid=(E // BE,),
        in_specs=[pl.BlockSpec((BE, H), lambda i: (i, 0)),
                  pl.BlockSpec((BE, FE), lambda i: (i, 0)),
                  pl.BlockSpec((FE, H), lambda i: (0, 0)),
                  pl.BlockSpec((1, H), lambda i: (0, 0))],
        out_specs=pl.BlockSpec((BE, H), lambda i: (i, 0)),
        out_shape=jax.ShapeDtypeStruct((E, H), jnp.float32),
    )(gath, ea, w2, b)


def _tc_mm_edge(s, w):
    def body(s_ref, w_ref, o_ref):
        o_ref[...] = jnp.dot(s_ref[...], w_ref[...],
                             preferred_element_type=jnp.float32)

    return pl.pallas_call(
        body,
        grid=(E // BE,),
        in_specs=[pl.BlockSpec((BE, H), lambda i: (i, 0)),
                  pl.BlockSpec((H, H), lambda i: (0, 0))],
        out_specs=pl.BlockSpec((BE, H), lambda i: (i, 0)),
        out_shape=jax.ShapeDtypeStruct((E, H), jnp.float32),
    )(s, w)


def _tc_sub_relu(gath, g):
    def body(a_ref, b_ref, o_ref):
        o_ref[...] = jnp.maximum(a_ref[...] - b_ref[...], 0.0)

    return pl.pallas_call(
        body,
        grid=(E // BE,),
        in_specs=[pl.BlockSpec((BE, H), lambda i: (i, 0)),
                  pl.BlockSpec((BE, H), lambda i: (i, 0))],
        out_specs=pl.BlockSpec((BE, H), lambda i: (i, 0)),
        out_shape=jax.ShapeDtypeStruct((E, H), jnp.float32),
    )(gath, g)


def _tc_final(x, p, w1, w2, b):
    def body(x_ref, p_ref, w1_ref, w2_ref, b_ref, o_ref):
        agg = p_ref[0] + p_ref[1]
        o_ref[...] = jnp.maximum(
            jnp.dot(x_ref[...], w1_ref[...], preferred_element_type=jnp.float32)
            + jnp.dot(agg, w2_ref[...], preferred_element_type=jnp.float32)
            + b_ref[...], 0.0)

    return pl.pallas_call(
        body,
        grid=(N // BN,),
        in_specs=[pl.BlockSpec((BN, FN), lambda i: (i, 0)),
                  pl.BlockSpec((NC, BN, H), lambda i: (0, i, 0)),
                  pl.BlockSpec((FN, H), lambda i: (0, 0)),
                  pl.BlockSpec((H, H), lambda i: (0, 0)),
                  pl.BlockSpec((1, H), lambda i: (0, 0))],
        out_specs=pl.BlockSpec((BN, H), lambda i: (i, 0)),
        out_shape=jax.ShapeDtypeStruct((N, H), jnp.float32),
    )(x, p, w1, w2, b)


def kernel(x, edge_index, edge_attr, W_init, b_init, W_convs, b_convs,
           W_e2n, b_e2n):
    depth = W_convs.shape[0]
    ei3 = jnp.pad(edge_index.reshape(2, CHUNKS, CH),
                  ((0, 0), (0, PADC), (0, 0)))
    eip3 = _tc_pairflip(ei3)
    zeros = jnp.zeros((NPT, H), jnp.float32)

    xw = _tc_mm_node(x, W_init[:FN])
    gath0 = _sc_gather(xw, ei3, 0)
    s = _tc_init_h(gath0, edge_attr, W_init[FN:], b_init.reshape(1, H))

    f = 0
    for i in range(depth):
        p = _sc_segsum(s, ei3 if f == 0 else eip3, 1, zeros)
        g = _tc_mm_edge(s, W_convs[i])
        a = _tc_aggw(p, W_convs[i], b_convs[i].reshape(1, H))
        f = 1 - f
        gath = _sc_gather(a, eip3 if f == 1 else ei3, 0)
        s = _tc_sub_relu(gath, g)

    p = _sc_segsum(s, ei3 if f == 0 else eip3, 1, zeros)
    return _tc_final(x, p, W_e2n[:FN], W_e2n[FN:], b_e2n.reshape(1, H))


# speedup vs baseline: 1.1049x; 1.1049x over previous
import functools

import jax
import jax.numpy as jnp
from jax import lax
from jax.experimental import pallas as pl
from jax.experimental.pallas import tpu as pltpu
from jax.experimental.pallas import tpu_sc as plsc

N, E, FN, FE, H = 10000, 320000, 128, 16, 128
NC, NS = 2, 16
NW = NC * NS
CH = 128
CHUNKS = E // CH
CPW = 80
LASTW = CHUNKS - (NW - 1) * CPW
PADC = 4
LASTW_LD = LASTW + PADC
NPAD = 10240
NPT = NPAD // NS
BE = 8000
BN = 1000
BNP = 1024
GG = 2
ROWS_G = GG * CH


def _sc_mesh():
    return plsc.VectorSubcoreMesh(core_axis_name="c", subcore_axis_name="s")


def _tc_pairflip(ei3):
    def body(x_ref, o_ref):
        x = x_ref[...]
        lane = lax.broadcasted_iota(jnp.int32, x.shape, 2)
        o_ref[...] = jnp.where((lane & 1) == 0,
                               pltpu.roll(x, CH - 1, 2),
                               pltpu.roll(x, 1, 2))

    return pl.pallas_call(
        body,
        out_shape=jax.ShapeDtypeStruct((2, CHUNKS + PADC, CH), jnp.int32),
    )(ei3)


def _sc_gather(table, idxarr, which):
    @functools.partial(
        pl.kernel,
        out_type=jax.ShapeDtypeStruct((E, H), jnp.float32),
        mesh=_sc_mesh(),
        scratch_types=[
            pltpu.VMEM((CPW, CH), jnp.int32),
            pltpu.VMEM((2, ROWS_G, H), jnp.float32),
            pltpu.SemaphoreType.DMA,
            pltpu.SemaphoreType.DMA,
            pltpu.SemaphoreType.DMA,
            pltpu.SemaphoreType.DMA,
        ],
    )
    def k(table_hbm, idx_hbm, out_hbm, idxs_v, bufs, sg0, sg1, ss0, ss1):
        cid = lax.axis_index("c")
        sid = lax.axis_index("s")
        wid = sid * NC + cid
        cb = wid * CPW
        base = cb * CH
        last = wid == NW - 1

        @pl.when(last)
        def _():
            pltpu.sync_copy(idx_hbm.at[which, pl.ds(cb, LASTW_LD)],
                            idxs_v.at[pl.ds(0, LASTW_LD)])

        @pl.when(jnp.logical_not(last))
        def _():
            pltpu.sync_copy(idx_hbm.at[which, pl.ds(cb, CPW)], idxs_v)

        half_groups = jnp.where(last, LASTW // GG // 2, CPW // GG // 2)
        ss = (ss0, ss1)
        sg = (sg0, sg1)

        def fire_gathers(g, s):
            buf = bufs.at[s]
            for b in range(GG):
                pltpu.async_copy(table_hbm.at[idxs_v.at[g * GG + b]],
                                 buf.at[pl.ds(b * CH, CH)], sg[s])

        def wait_and_store(g, s):
            buf = bufs.at[s]
            for b in range(GG):
                pltpu.make_async_copy(table_hbm.at[idxs_v.at[0]],
                                      buf.at[pl.ds(b * CH, CH)], sg[s]).wait()
            pltpu.async_copy(buf, out_hbm.at[pl.ds(base + g * ROWS_G, ROWS_G)],
                             ss[s])

        def drain_store(s):
            pltpu.make_async_copy(bufs.at[s],
                                  out_hbm.at[pl.ds(base, ROWS_G)], ss[s]).wait()

        def body(i, carry):
            for s in range(2):
                @pl.when(i >= 1)
                def _():
                    drain_store(s)

                fire_gathers(2 * i + s, s)
            for s in range(2):
                wait_and_store(2 * i + s, s)
            return carry

        lax.fori_loop(0, half_groups, body, 0)
        drain_store(0)
        drain_store(1)

    return k(table, idxarr)


def _sc_segsum(h, idxarr, which, zeros):
    @functools.partial(
        pl.kernel,
        out_type=jax.ShapeDtypeStruct((NC, NPAD, H), jnp.float32),
        mesh=_sc_mesh(),
        scratch_types=[
            pltpu.VMEM((CPW, CH), jnp.int32),
            pltpu.VMEM((2, CH, H), jnp.float32),
            pltpu.VMEM_SHARED((NPAD, H), jnp.float32),
            pltpu.SemaphoreType.DMA,
            pltpu.SemaphoreType.DMA,
            pltpu.SemaphoreType.DMA,
            pltpu.SemaphoreType.DMA,
        ],
    )
    def k(h_hbm, idx_hbm, z_hbm, out_hbm, idxs_v, bufs, acc, sl0, sl1,
          sa0, sa1):
        cid = lax.axis_index("c")
        sid = lax.axis_index("s")
        wid = sid * NC + cid
        cb = wid * CPW
        base = cb * CH
        last = wid == NW - 1
        pltpu.sync_copy(z_hbm, acc.at[pl.ds(sid * NPT, NPT)])

        @pl.when(last)
        def _():
            pltpu.sync_copy(idx_hbm.at[which, pl.ds(cb, LASTW_LD)],
                            idxs_v.at[pl.ds(0, LASTW_LD)])

        @pl.when(jnp.logical_not(last))
        def _():
            pltpu.sync_copy(idx_hbm.at[which, pl.ds(cb, CPW)], idxs_v)

        half_chunks = jnp.where(last, LASTW // 2, CPW // 2)
        plsc.subcore_barrier()
        sa = (sa0, sa1)
        sl = (sl0, sl1)

        def drain_add(s):
            pltpu.make_async_copy(bufs.at[s], acc.at[idxs_v.at[0]],
                                  sa[s]).wait()

        def body(i, carry):
            for s in range(2):
                @pl.when(i >= 1)
                def _():
                    drain_add(s)

                pltpu.async_copy(h_hbm.at[pl.ds(base + (2 * i + s) * CH, CH)],
                                 bufs.at[s], sl[s])
            for s in range(2):
                pltpu.make_async_copy(h_hbm.at[pl.ds(base, CH)], bufs.at[s],
                                      sl[s]).wait()
                pltpu.async_copy(bufs.at[s], acc.at[idxs_v.at[2 * i + s]],
                                 sa[s], add=True)
            return carry

        lax.fori_loop(0, half_chunks, body, 0)
        drain_add(0)
        drain_add(1)
        plsc.subcore_barrier()
        pltpu.sync_copy(acc.at[pl.ds(sid * NPT, NPT)],
                        out_hbm.at[cid, pl.ds(sid * NPT, NPT)])

    return k(h, idxarr, zeros)


def _tc_mm_node(x, w):
    def body(x_ref, w_ref, o_ref):
        o_ref[...] = jnp.dot(x_ref[...], w_ref[...],
                             preferred_element_type=jnp.float32)

    return pl.pallas_call(
        body,
        grid=(N // BN,),
        in_specs=[pl.BlockSpec((BN, FN), lambda i: (i, 0)),
                  pl.BlockSpec((FN, H), lambda i: (0, 0))],
        out_specs=pl.BlockSpec((BN, H), lambda i: (i, 0)),
        out_shape=jax.ShapeDtypeStruct((N, H), jnp.float32),
    )(x, w)


def _tc_aggw(p, w, b):
    def body(p_ref, w_ref, b_ref, o_ref):
        agg = p_ref[0] + p_ref[1]
        o_ref[...] = jnp.dot(agg, w_ref[...],
                             preferred_element_type=jnp.float32) + b_ref[...]

    return pl.pallas_call(
        body,
        grid=(NPAD // BNP,),
        in_specs=[pl.BlockSpec((NC, BNP, H), lambda i: (0, i, 0)),
                  pl.BlockSpec((H, H), lambda i: (0, 0)),
                  pl.BlockSpec((1, H), lambda i: (0, 0))],
        out_specs=pl.BlockSpec((BNP, H), lambda i: (i, 0)),
        out_shape=jax.ShapeDtypeStruct((NPAD, H), jnp.float32),
    )(p, w, b)


def _tc_init_h(gath, ea, w2, b):
    def body(g_ref, e_ref, w_ref, b_ref, o_ref):
        o_ref[...] = jnp.maximum(
            g_ref[...] + jnp.dot(e_ref[...], w_ref[...],
                                 preferred_element_type=jnp.float32)
            + b_ref[...], 0.0)

    return pl.pallas_call(
        body,
        grid=(E // BE,),
        in_specs=[pl.BlockSpec((BE, H), lambda i: (i, 0)),
                  pl.BlockSpec((BE, FE), lambda i: (i, 0)),
                  pl.BlockSpec((FE, H), lambda i: (0, 0)),
                  pl.BlockSpec((1, H), lambda i: (0, 0))],
        out_specs=pl.BlockSpec((BE, H), lambda i: (i, 0)),
        out_shape=jax.ShapeDtypeStruct((E, H), jnp.float32),
    )(gath, ea, w2, b)


def _tc_conv_update(gath, s, w):
    def body(g_ref, s_ref, w_ref, o_ref):
        o_ref[...] = jnp.maximum(
            g_ref[...] - jnp.dot(s_ref[...], w_ref[...],
                                 preferred_element_type=jnp.float32), 0.0)

    return pl.pallas_call(
        body,
        grid=(E // BE,),
        in_specs=[pl.BlockSpec((BE, H), lambda i: (i, 0)),
                  pl.BlockSpec((BE, H), lambda i: (i, 0)),
                  pl.BlockSpec((H, H), lambda i: (0, 0))],
        out_specs=pl.BlockSpec((BE, H), lambda i: (i, 0)),
        out_shape=jax.ShapeDtypeStruct((E, H), jnp.float32),
    )(gath, s, w)


def _tc_final(x, p, w1, w2, b):
    def body(x_ref, p_ref, w1_ref, w2_ref, b_ref, o_ref):
        agg = p_ref[0] + p_ref[1]
        o_ref[...] = jnp.maximum(
            jnp.dot(x_ref[...], w1_ref[...], preferred_element_type=jnp.float32)
            + jnp.dot(agg, w2_ref[...], preferred_element_type=jnp.float32)
            + b_ref[...], 0.0)

    return pl.pallas_call(
        body,
        grid=(N // BN,),
        in_specs=[pl.BlockSpec((BN, FN), lambda i: (i, 0)),
                  pl.BlockSpec((NC, BN, H), lambda i: (0, i, 0)),
                  pl.BlockSpec((FN, H), lambda i: (0, 0)),
                  pl.BlockSpec((H, H), lambda i: (0, 0)),
                  pl.BlockSpec((1, H), lambda i: (0, 0))],
        out_specs=pl.BlockSpec((BN, H), lambda i: (i, 0)),
        out_shape=jax.ShapeDtypeStruct((N, H), jnp.float32),
    )(x, p, w1, w2, b)


def kernel(x, edge_index, edge_attr, W_init, b_init, W_convs, b_convs,
           W_e2n, b_e2n):
    depth = W_convs.shape[0]
    ei3 = jnp.pad(edge_index.reshape(2, CHUNKS, CH),
                  ((0, 0), (0, PADC), (0, 0)))
    eip3 = _tc_pairflip(ei3)
    zeros = jnp.zeros((NPT, H), jnp.float32)

    xw = _tc_mm_node(x, W_init[:FN])
    gath0 = _sc_gather(xw, ei3, 0)
    s = _tc_init_h(gath0, edge_attr, W_init[FN:], b_init.reshape(1, H))

    f = 0
    for i in range(depth):
        p = _sc_segsum(s, ei3 if f == 0 else eip3, 1, zeros)
        a = _tc_aggw(p, W_convs[i], b_convs[i].reshape(1, H))
        f = 1 - f
        gath = _sc_gather(a, eip3 if f == 1 else ei3, 0)
        s = _tc_conv_update(gath, s, W_convs[i])

    p = _sc_segsum(s, ei3 if f == 0 else eip3, 1, zeros)
    return _tc_final(x, p, W_e2n[:FN], W_e2n[FN:], b_e2n.reshape(1, H))


# speedup vs baseline: 1.1593x; 1.0493x over previous
import functools

import jax
import jax.numpy as jnp
from jax import lax
from jax.experimental import pallas as pl
from jax.experimental.pallas import tpu as pltpu
from jax.experimental.pallas import tpu_sc as plsc

N, E, FN, FE, H = 10000, 320000, 128, 16, 128
NC, NS = 2, 16
NW = NC * NS
CH = 128
CHUNKS = E // CH
CPW = 80
LASTW = CHUNKS - (NW - 1) * CPW
PADC = 4
LASTW_LD = LASTW + PADC
NPAD = 10240
NPT = NPAD // NS
BE = 16000
BN = 1000
BNP = 1024
GG = 2
ROWS_G = GG * CH


def _sc_mesh():
    return plsc.VectorSubcoreMesh(core_axis_name="c", subcore_axis_name="s")


def _tc_pairflip(ei3):
    def body(x_ref, o_ref):
        x = x_ref[...]
        lane = lax.broadcasted_iota(jnp.int32, x.shape, 2)
        o_ref[...] = jnp.where((lane & 1) == 0,
                               pltpu.roll(x, CH - 1, 2),
                               pltpu.roll(x, 1, 2))

    return pl.pallas_call(
        body,
        out_shape=jax.ShapeDtypeStruct((2, CHUNKS + PADC, CH), jnp.int32),
    )(ei3)


def _sc_gather(table, idxarr, which):
    @functools.partial(
        pl.kernel,
        out_type=jax.ShapeDtypeStruct((E, H), jnp.float32),
        mesh=_sc_mesh(),
        scratch_types=[
            pltpu.VMEM((CPW, CH), jnp.int32),
            pltpu.VMEM((2, ROWS_G, H), jnp.float32),
            pltpu.SemaphoreType.DMA,
            pltpu.SemaphoreType.DMA,
            pltpu.SemaphoreType.DMA,
        ],
    )
    def k(table_hbm, idx_hbm, out_hbm, idxs_v, bufs, sg, ss0, ss1):
        cid = lax.axis_index("c")
        sid = lax.axis_index("s")
        wid = sid * NC + cid
        cb = wid * CPW
        base = cb * CH
        last = wid == NW - 1

        @pl.when(last)
        def _():
            pltpu.sync_copy(idx_hbm.at[which, pl.ds(cb, LASTW_LD)],
                            idxs_v.at[pl.ds(0, LASTW_LD)])

        @pl.when(jnp.logical_not(last))
        def _():
            pltpu.sync_copy(idx_hbm.at[which, pl.ds(cb, CPW)], idxs_v)

        half_groups = jnp.where(last, LASTW // GG // 2, CPW // GG // 2)
        ss = (ss0, ss1)

        def run_group(g, s):
            buf = bufs.at[s]
            for b in range(GG):
                pltpu.async_copy(table_hbm.at[idxs_v.at[g * GG + b]],
                                 buf.at[pl.ds(b * CH, CH)], sg)
            for b in range(GG):
                pltpu.make_async_copy(table_hbm.at[idxs_v.at[0]],
                                      buf.at[pl.ds(b * CH, CH)], sg).wait()
            pltpu.async_copy(buf, out_hbm.at[pl.ds(base + g * ROWS_G, ROWS_G)],
                             ss[s])

        def drain_store(s):
            pltpu.make_async_copy(bufs.at[s],
                                  out_hbm.at[pl.ds(base, ROWS_G)], ss[s]).wait()

        def body(i, carry):
            for s in range(2):
                @pl.when(i >= 1)
                def _():
                    drain_store(s)

                run_group(2 * i + s, s)
            return carry

        lax.fori_loop(0, half_groups, body, 0)
        drain_store(0)
        drain_store(1)

    return k(table, idxarr)


def _sc_segsum(h, idxarr, which, zeros):
    @functools.partial(
        pl.kernel,
        out_type=jax.ShapeDtypeStruct((NC, NPAD, H), jnp.float32),
        mesh=_sc_mesh(),
        scratch_types=[
            pltpu.VMEM((CPW, CH), jnp.int32),
            pltpu.VMEM((2, CH, H), jnp.float32),
            pltpu.VMEM_SHARED((NPAD, H), jnp.float32),
            pltpu.SemaphoreType.DMA,
            pltpu.SemaphoreType.DMA,
            pltpu.SemaphoreType.DMA,
        ],
    )
    def k(h_hbm, idx_hbm, z_hbm, out_hbm, idxs_v, bufs, acc, sl, sa0, sa1):
        cid = lax.axis_index("c")
        sid = lax.axis_index("s")
        wid = sid * NC + cid
        cb = wid * CPW
        base = cb * CH
        last = wid == NW - 1
        pltpu.sync_copy(z_hbm, acc.at[pl.ds(sid * NPT, NPT)])

        @pl.when(last)
        def _():
            pltpu.sync_copy(idx_hbm.at[which, pl.ds(cb, LASTW_LD)],
                            idxs_v.at[pl.ds(0, LASTW_LD)])

        @pl.when(jnp.logical_not(last))
        def _():
            pltpu.sync_copy(idx_hbm.at[which, pl.ds(cb, CPW)], idxs_v)

        half_chunks = jnp.where(last, LASTW // 2, CPW // 2)
        plsc.subcore_barrier()
        sa = (sa0, sa1)

        def drain_add(s):
            pltpu.make_async_copy(bufs.at[s], acc.at[idxs_v.at[0]],
                                  sa[s]).wait()

        def run_chunk(j, s):
            buf = bufs.at[s]
            pltpu.async_copy(h_hbm.at[pl.ds(base + j * CH, CH)], buf, sl)
            pltpu.make_async_copy(h_hbm.at[pl.ds(base, CH)], buf, sl).wait()
            pltpu.async_copy(buf, acc.at[idxs_v.at[j]], sa[s], add=True)

        def body(i, carry):
            for s in range(2):
                @pl.when(i >= 1)
                def _():
                    drain_add(s)

                run_chunk(2 * i + s, s)
            return carry

        lax.fori_loop(0, half_chunks, body, 0)
        drain_add(0)
        drain_add(1)
        plsc.subcore_barrier()
        pltpu.sync_copy(acc.at[pl.ds(sid * NPT, NPT)],
                        out_hbm.at[cid, pl.ds(sid * NPT, NPT)])

    return k(h, idxarr, zeros)


def _tc_mm_node(x, w):
    def body(x_ref, w_ref, o_ref):
        o_ref[...] = jnp.dot(x_ref[...], w_ref[...],
                             preferred_element_type=jnp.float32)

    return pl.pallas_call(
        body,
        grid=(N // BN,),
        in_specs=[pl.BlockSpec((BN, FN), lambda i: (i, 0)),
                  pl.BlockSpec((FN, H), lambda i: (0, 0))],
        out_specs=pl.BlockSpec((BN, H), lambda i: (i, 0)),
        out_shape=jax.ShapeDtypeStruct((N, H), jnp.float32),
    )(x, w)


def _tc_aggw(p, w, b):
    def body(p_ref, w_ref, b_ref, o_ref):
        agg = p_ref[0] + p_ref[1]
        o_ref[...] = jnp.dot(agg, w_ref[...],
                             preferred_element_type=jnp.float32) + b_ref[...]

    return pl.pallas_call(
        body,
        grid=(NPAD // BNP,),
        in_specs=[pl.BlockSpec((NC, BNP, H), lambda i: (0, i, 0)),
                  pl.BlockSpec((H, H), lambda i: (0, 0)),
                  pl.BlockSpec((1, H), lambda i: (0, 0))],
        out_specs=pl.BlockSpec((BNP, H), lambda i: (i, 0)),
        out_shape=jax.ShapeDtypeStruct((NPAD, H), jnp.float32),
    )(p, w, b)


def _tc_init_h(gath, ea, w2, b):
    def body(g_ref, e_ref, w_ref, b_ref, o_ref):
        o_ref[...] = jnp.maximum(
            g_ref[...] + jnp.dot(e_ref[...], w_ref[...],
                                 preferred_element_type=jnp.float32)
            + b_ref[...], 0.0)

    return pl.pallas_call(
        body,
        grid=(E // BE,),
        in_specs=[pl.BlockSpec((BE, H), lambda i: (i, 0)),
                  pl.BlockSpec((BE, FE), lambda i: (i, 0)),
                  pl.BlockSpec((FE, H), lambda i: (0, 0)),
                  pl.BlockSpec((1, H), lambda i: (0, 0))],
        out_specs=pl.BlockSpec((BE, H), lambda i: (i, 0)),
        out_shape=jax.ShapeDtypeStruct((E, H), jnp.float32),
    )(gath, ea, w2, b)


def _tc_conv_update(gath, s, w):
    def body(g_ref, s_ref, w_ref, o_ref):
        o_ref[...] = jnp.maximum(
            g_ref[...] - jnp.dot(s_ref[...], w_ref[...],
                                 preferred_element_type=jnp.float32), 0.0)

    return pl.pallas_call(
        body,
        grid=(E // BE,),
        in_specs=[pl.BlockSpec((BE, H), lambda i: (i, 0)),
                  pl.BlockSpec((BE, H), lambda i: (i, 0)),
                  pl.BlockSpec((H, H), lambda i: (0, 0))],
        out_specs=pl.BlockSpec((BE, H), lambda i: (i, 0)),
        out_shape=jax.ShapeDtypeStruct((E, H), jnp.float32),
    )(gath, s, w)


def _tc_final(x, p, w1, w2, b):
    def body(x_ref, p_ref, w1_ref, w2_ref, b_ref, o_ref):
        agg = p_ref[0] + p_ref[1]
        o_ref[...] = jnp.maximum(
            jnp.dot(x_ref[...], w1_ref[...], preferred_element_type=jnp.float32)
            + jnp.dot(agg, w2_ref[...], preferred_element_type=jnp.float32)
            + b_ref[...], 0.0)

    return pl.pallas_call(
        body,
        grid=(N // BN,),
        in_specs=[pl.BlockSpec((BN, FN), lambda i: (i, 0)),
                  pl.BlockSpec((NC, BN, H), lambda i: (0, i, 0)),
                  pl.BlockSpec((FN, H), lambda i: (0, 0)),
                  pl.BlockSpec((H, H), lambda i: (0, 0)),
                  pl.BlockSpec((1, H), lambda i: (0, 0))],
        out_specs=pl.BlockSpec((BN, H), lambda i: (i, 0)),
        out_shape=jax.ShapeDtypeStruct((N, H), jnp.float32),
    )(x, p, w1, w2, b)


def kernel(x, edge_index, edge_attr, W_init, b_init, W_convs, b_convs,
           W_e2n, b_e2n):
    depth = W_convs.shape[0]
    ei3 = jnp.pad(edge_index.reshape(2, CHUNKS, CH),
                  ((0, 0), (0, PADC), (0, 0)))
    eip3 = _tc_pairflip(ei3)
    zeros = jnp.zeros((NPT, H), jnp.float32)

    xw = _tc_mm_node(x, W_init[:FN])
    gath0 = _sc_gather(xw, ei3, 0)
    s = _tc_init_h(gath0, edge_attr, W_init[FN:], b_init.reshape(1, H))

    f = 0
    for i in range(depth):
        p = _sc_segsum(s, ei3 if f == 0 else eip3, 1, zeros)
        a = _tc_aggw(p, W_convs[i], b_convs[i].reshape(1, H))
        f = 1 - f
        gath = _sc_gather(a, eip3 if f == 1 else ei3, 0)
        s = _tc_conv_update(gath, s, W_convs[i])

    p = _sc_segsum(s, ei3 if f == 0 else eip3, 1, zeros)
    return _tc_final(x, p, W_e2n[:FN], W_e2n[FN:], b_e2n.reshape(1, H))


# speedup vs baseline: 1.1635x; 1.0037x over previous
import functools

import jax
import jax.numpy as jnp
from jax import lax
from jax.experimental import pallas as pl
from jax.experimental.pallas import tpu as pltpu
from jax.experimental.pallas import tpu_sc as plsc

N, E, FN, FE, H = 10000, 320000, 128, 16, 128
NC, NS = 2, 16
NW = NC * NS
CH = 128
CHUNKS = E // CH
CPW = 80
LASTW = CHUNKS - (NW - 1) * CPW
PADC = 4
LASTW_LD = LASTW + PADC
NPAD = 10240
NPT = NPAD // NS
BE = 8000
BN = 1000
BNP = 1024
GG = 2
ROWS_G = GG * CH


def _sc_mesh():
    return plsc.VectorSubcoreMesh(core_axis_name="c", subcore_axis_name="s")


def _tc_pairflip(ei3):
    def body(x_ref, o_ref):
        x = x_ref[...]
        lane = lax.broadcasted_iota(jnp.int32, x.shape, 2)
        o_ref[...] = jnp.where((lane & 1) == 0,
                               pltpu.roll(x, CH - 1, 2),
                               pltpu.roll(x, 1, 2))

    return pl.pallas_call(
        body,
        out_shape=jax.ShapeDtypeStruct((2, CHUNKS + PADC, CH), jnp.int32),
    )(ei3)


def _sc_gather(table, idxarr, which):
    @functools.partial(
        pl.kernel,
        out_type=jax.ShapeDtypeStruct((E, H), jnp.float32),
        mesh=_sc_mesh(),
        scratch_types=[
            pltpu.VMEM((CPW, CH), jnp.int32),
            pltpu.VMEM((2, ROWS_G, H), jnp.float32),
            pltpu.SemaphoreType.DMA,
            pltpu.SemaphoreType.DMA,
            pltpu.SemaphoreType.DMA,
        ],
    )
    def k(table_hbm, idx_hbm, out_hbm, idxs_v, bufs, sg, ss0, ss1):
        cid = lax.axis_index("c")
        sid = lax.axis_index("s")
        wid = sid * NC + cid
        cb = wid * CPW
        base = cb * CH
        last = wid == NW - 1

        @pl.when(last)
        def _():
            pltpu.sync_copy(idx_hbm.at[which, pl.ds(cb, LASTW_LD)],
                            idxs_v.at[pl.ds(0, LASTW_LD)])

        @pl.when(jnp.logical_not(last))
        def _():
            pltpu.sync_copy(idx_hbm.at[which, pl.ds(cb, CPW)], idxs_v)

        half_groups = jnp.where(last, LASTW // GG // 2, CPW // GG // 2)
        ss = (ss0, ss1)

        def run_group(g, s):
            buf = bufs.at[s]
            for b in range(GG):
                pltpu.async_copy(table_hbm.at[idxs_v.at[g * GG + b]],
                                 buf.at[pl.ds(b * CH, CH)], sg)
            for b in range(GG):
                pltpu.make_async_copy(table_hbm.at[idxs_v.at[0]],
                                      buf.at[pl.ds(b * CH, CH)], sg).wait()
            pltpu.async_copy(buf, out_hbm.at[pl.ds(base + g * ROWS_G, ROWS_G)],
                             ss[s])

        def drain_store(s):
            pltpu.make_async_copy(bufs.at[s],
                                  out_hbm.at[pl.ds(base, ROWS_G)], ss[s]).wait()

        def body(i, carry):
            for s in range(2):
                @pl.when(i >= 1)
                def _():
                    drain_store(s)

                run_group(2 * i + s, s)
            return carry

        lax.fori_loop(0, half_groups, body, 0)
        drain_store(0)
        drain_store(1)

    return k(table, idxarr)


def _sc_segsum(h, idxarr, which, zeros):
    @functools.partial(
        pl.kernel,
        out_type=jax.ShapeDtypeStruct((NC, NPAD, H), jnp.float32),
        mesh=_sc_mesh(),
        scratch_types=[
            pltpu.VMEM((CPW, CH), jnp.int32),
            pltpu.VMEM((2, CH, H), jnp.float32),
            pltpu.VMEM_SHARED((NPAD, H), jnp.float32),
            pltpu.SemaphoreType.DMA,
            pltpu.SemaphoreType.DMA,
            pltpu.SemaphoreType.DMA,
        ],
    )
    def k(h_hbm, idx_hbm, z_hbm, out_hbm, idxs_v, bufs, acc, sl, sa0, sa1):
        cid = lax.axis_index("c")
        sid = lax.axis_index("s")
        wid = sid * NC + cid
        cb = wid * CPW
        base = cb * CH
        last = wid == NW - 1
        pltpu.sync_copy(z_hbm, acc.at[pl.ds(sid * NPT, NPT)])

        @pl.when(last)
        def _():
            pltpu.sync_copy(idx_hbm.at[which, pl.ds(cb, LASTW_LD)],
                            idxs_v.at[pl.ds(0, LASTW_LD)])

        @pl.when(jnp.logical_not(last))
        def _():
            pltpu.sync_copy(idx_hbm.at[which, pl.ds(cb, CPW)], idxs_v)

        half_chunks = jnp.where(last, LASTW // 2, CPW // 2)
        plsc.subcore_barrier()
        sa = (sa0, sa1)

        def drain_add(s):
            pltpu.make_async_copy(bufs.at[s], acc.at[idxs_v.at[0]],
                                  sa[s]).wait()

        def run_chunk(j, s):
            buf = bufs.at[s]
            pltpu.async_copy(h_hbm.at[pl.ds(base + j * CH, CH)], buf, sl)
            pltpu.make_async_copy(h_hbm.at[pl.ds(base, CH)], buf, sl).wait()
            pltpu.async_copy(buf, acc.at[idxs_v.at[j]], sa[s], add=True)

        def body(i, carry):
            for s in range(2):
                @pl.when(i >= 1)
                def _():
                    drain_add(s)

                run_chunk(2 * i + s, s)
            return carry

        lax.fori_loop(0, half_chunks, body, 0)
        drain_add(0)
        drain_add(1)
        plsc.subcore_barrier()
        pltpu.sync_copy(acc.at[pl.ds(sid * NPT, NPT)],
                        out_hbm.at[cid, pl.ds(sid * NPT, NPT)])

    return k(h, idxarr, zeros)


def _tc_mm_node(x, w):
    def body(x_ref, w_ref, o_ref):
        o_ref[...] = jnp.dot(x_ref[...], w_ref[...],
                             preferred_element_type=jnp.float32)

    return pl.pallas_call(
        body,
        grid=(N // BN,),
        in_specs=[pl.BlockSpec((BN, FN), lambda i: (i, 0)),
                  pl.BlockSpec((FN, H), lambda i: (0, 0))],
        out_specs=pl.BlockSpec((BN, H), lambda i: (i, 0)),
        out_shape=jax.ShapeDtypeStruct((N, H), jnp.float32),
    )(x, w)


def _tc_aggw(p, w, b):
    def body(p_ref, w_ref, b_ref, o_ref):
        agg = p_ref[0] + p_ref[1]
        o_ref[...] = jnp.dot(agg, w_ref[...],
                             preferred_element_type=jnp.float32) + b_ref[...]

    return pl.pallas_call(
        body,
        grid=(NPAD // BNP,),
        in_specs=[pl.BlockSpec((NC, BNP, H), lambda i: (0, i, 0)),
                  pl.BlockSpec((H, H), lambda i: (0, 0)),
                  pl.BlockSpec((1, H), lambda i: (0, 0))],
        out_specs=pl.BlockSpec((BNP, H), lambda i: (i, 0)),
        out_shape=jax.ShapeDtypeStruct((NPAD, H), jnp.float32),
    )(p, w, b)


def _tc_init_h(gath, ea, w2, b):
    def body(g_ref, e_ref, w_ref, b_ref, o_ref):
        o_ref[...] = jnp.maximum(
            g_ref[...] + jnp.dot(e_ref[...], w_ref[...],
                                 preferred_element_type=jnp.float32)
            + b_ref[...], 0.0)

    return pl.pallas_call(
        body,
        grid=(E // BE,),
        in_specs=[pl.BlockSpec((BE, H), lambda i: (i, 0)),
                  pl.BlockSpec((BE, FE), lambda i: (i, 0)),
                  pl.BlockSpec((FE, H), lambda i: (0, 0)),
                  pl.BlockSpec((1, H), lambda i: (0, 0))],
        out_specs=pl.BlockSpec((BE, H), lambda i: (i, 0)),
        out_shape=jax.ShapeDtypeStruct((E, H), jnp.float32),
    )(gath, ea, w2, b)


def _tc_conv_update(gath, s, w):
    def body(g_ref, s_ref, w_ref, o_ref):
        o_ref[...] = jnp.maximum(
            g_ref[...] - jnp.dot(s_ref[...], w_ref[...],
                                 preferred_element_type=jnp.float32), 0.0)

    return pl.pallas_call(
        body,
        grid=(E // BE,),
        in_specs=[pl.BlockSpec((BE, H), lambda i: (i, 0)),
                  pl.BlockSpec((BE, H), lambda i: (i, 0)),
                  pl.BlockSpec((H, H), lambda i: (0, 0))],
        out_specs=pl.BlockSpec((BE, H), lambda i: (i, 0)),
        out_shape=jax.ShapeDtypeStruct((E, H), jnp.float32),
    )(gath, s, w)


def _tc_final(x, p, w1, w2, b):
    def body(x_ref, p_ref, w1_ref, w2_ref, b_ref, o_ref):
        agg = p_ref[0] + p_ref[1]
        o_ref[...] = jnp.maximum(
            jnp.dot(x_ref[...], w1_ref[...], preferred_element_type=jnp.float32)
            + jnp.dot(agg, w2_ref[...], preferred_element_type=jnp.float32)
            + b_ref[...], 0.0)

    return pl.pallas_call(
        body,
        grid=(N // BN,),
        in_specs=[pl.BlockSpec((BN, FN), lambda i: (i, 0)),
                  pl.BlockSpec((NC, BN, H), lambda i: (0, i, 0)),
                  pl.BlockSpec((FN, H), lambda i: (0, 0)),
                  pl.BlockSpec((H, H), lambda i: (0, 0)),
                  pl.BlockSpec((1, H), lambda i: (0, 0))],
        out_specs=pl.BlockSpec((BN, H), lambda i: (i, 0)),
        out_shape=jax.ShapeDtypeStruct((N, H), jnp.float32),
    )(x, p, w1, w2, b)


def kernel(x, edge_index, edge_attr, W_init, b_init, W_convs, b_convs,
           W_e2n, b_e2n):
    depth = W_convs.shape[0]
    ei3 = jnp.pad(edge_index.reshape(2, CHUNKS, CH),
                  ((0, 0), (0, PADC), (0, 0)))
    eip3 = _tc_pairflip(ei3)
    zeros = jnp.zeros((NPT, H), jnp.float32)

    xw = _tc_mm_node(x, W_init[:FN])
    gath0 = _sc_gather(xw, ei3, 0)
    s = _tc_init_h(gath0, edge_attr, W_init[FN:], b_init.reshape(1, H))

    f = 0
    for i in range(depth):
        p = _sc_segsum(s, ei3 if f == 0 else eip3, 1, zeros)
        a = _tc_aggw(p, W_convs[i], b_convs[i].reshape(1, H))
        f = 1 - f
        gath = _sc_gather(a, eip3 if f == 1 else ei3, 0)
        s = _tc_conv_update(gath, s, W_convs[i])

    p = _sc_segsum(s, ei3 if f == 0 else eip3, 1, zeros)
    return _tc_final(x, p, W_e2n[:FN], W_e2n[FN:], b_e2n.reshape(1, H))
